# Initial kernel scaffold; baseline (speedup 1.0000x reference)
import jax
import jax.numpy as jnp
from jax.experimental import pallas as pl


def kernel(index, length, data):
    raise NotImplementedError("write your pallas kernel here")


# speedup vs baseline: 1.6940x; 1.6940x over previous
import functools

import jax
import jax.numpy as jnp
from jax import lax
from jax.experimental import pallas as pl
from jax.experimental.pallas import tpu as pltpu
from jax.experimental.pallas import tpu_sc as plsc

CYCLE_LEN = 168
CHANNEL_SIZE = 321
BATCH = 4096
LENGTH = 96

_NUM_CORES = 2
_NUM_SUBCORES = 16
_NUM_WORKERS = _NUM_CORES * _NUM_SUBCORES
_PER_WORKER = BATCH // _NUM_WORKERS
_GROUPS = _PER_WORKER // 16


@functools.partial(
    pl.kernel,
    out_type=jax.ShapeDtypeStruct((BATCH, LENGTH, CHANNEL_SIZE), jnp.float32),
    mesh=plsc.VectorSubcoreMesh(core_axis_name="c", subcore_axis_name="s"),
    compiler_params=pltpu.CompilerParams(use_tc_tiling_on_sc=False),
    scratch_types=[
        pltpu.VMEM((2 * CYCLE_LEN, CHANNEL_SIZE), jnp.float32),
        pltpu.VMEM((_PER_WORKER,), jnp.int32),
        pltpu.SemaphoreType.DMA,
    ],
)
def _cycle_gather(starts_hbm, data_hbm, out_hbm, tab_v, idx_v, sem):
    wid = lax.axis_index("s") * _NUM_CORES + lax.axis_index("c")
    base = wid * _PER_WORKER

    pltpu.sync_copy(data_hbm, tab_v.at[pl.ds(0, CYCLE_LEN), :])
    pltpu.sync_copy(data_hbm, tab_v.at[pl.ds(CYCLE_LEN, CYCLE_LEN), :])
    pltpu.sync_copy(starts_hbm.at[pl.ds(base, _PER_WORKER)], idx_v)

    def group(g, carry):
        vec = idx_v[pl.ds(g * 16, 16)]
        for l in range(16):
            start = vec[l]
            i = base + g * 16 + l
            pltpu.async_copy(tab_v.at[pl.ds(start, LENGTH), :], out_hbm.at[i], sem)
        return carry

    lax.fori_loop(0, _GROUPS, group, 0)

    pltpu.make_async_copy(
        out_hbm.at[pl.ds(base, _PER_WORKER)],
        out_hbm.at[pl.ds(base, _PER_WORKER)],
        sem,
    ).wait()


def kernel(index, length, data):
    shift = jnp.asarray(length, jnp.int32) - LENGTH
    starts = (index.astype(jnp.int32) + shift) % CYCLE_LEN
    return _cycle_gather(starts, data)
